# Initial kernel scaffold; baseline (speedup 1.0000x reference)
import functools

import jax
import jax.numpy as jnp
from jax import lax
from jax.experimental import pallas as pl
from jax.experimental.pallas import tpu as pltpu
from jax.experimental.pallas import tpu_sc as plsc


def _normalize_body(x_ref, o_ref):
    v = x_ref[...]
    n = jnp.sqrt(jnp.sum(v * v, axis=1, keepdims=True))
    o_ref[...] = v / jnp.maximum(n, 1e-12)


def _normalize_tc(x):
    n, d = x.shape
    return pl.pallas_call(
        _normalize_body,
        out_shape=jax.ShapeDtypeStruct((n, d), jnp.float32),
    )(x)


_L = 16


def _make_edge_dot(n_nodes, d, e, nw):
    epw = e // nw
    c = 80
    assert epw % c == 0 and c % _L == 0 and c % 8 == 0
    n_chunks = epw // c
    groups = c // _L

    mesh = plsc.VectorSubcoreMesh(core_axis_name="c", subcore_axis_name="s")
    nc = 2

    @functools.partial(
        pl.kernel,
        mesh=mesh,
        out_type=jax.ShapeDtypeStruct((e,), jnp.float32),
        scratch_types=[
            pltpu.VMEM((c,), jnp.int32),
            pltpu.VMEM((c,), jnp.int32),
            pltpu.VMEM((c, d), jnp.float32),
            pltpu.VMEM((c, d), jnp.float32),
            pltpu.VMEM((c,), jnp.float32),
            pltpu.SemaphoreType.DMA,
            pltpu.SemaphoreType.DMA,
        ],
    )
    def edge_dot(h_hbm, src_hbm, dst_hbm, out_hbm,
                 si_v, di_v, ru_v, rv_v, oc_v, sem_u, sem_v):
        wid = lax.axis_index("s") * nc + lax.axis_index("c")
        wbase = wid * epw

        def chunk_body(ci, carry):
            base = wbase + ci * c
            pltpu.sync_copy(src_hbm.at[pl.ds(base, c)], si_v)
            pltpu.sync_copy(dst_hbm.at[pl.ds(base, c)], di_v)
            cu = pltpu.async_copy(h_hbm.at[si_v], ru_v, sem_u)
            cv = pltpu.async_copy(h_hbm.at[di_v], rv_v, sem_v)
            cu.wait()
            cv.wait()

            lane = lax.broadcasted_iota(jnp.int32, (_L,), 0)

            def grp_body(g, carry2):
                rows = lane + g * _L
                acc = jnp.zeros((_L,), jnp.float32)

                def feat_body(jb, acc):
                    for jj in range(8):
                        cols = jnp.full((_L,), jb * 8 + jj, dtype=jnp.int32)
                        gu = plsc.load_gather(ru_v, [rows, cols])
                        gv = plsc.load_gather(rv_v, [rows, cols])
                        acc = acc + gu * gv
                    return acc

                acc = lax.fori_loop(0, d // 8, feat_body, acc)
                oc_v[pl.ds(g * _L, _L)] = acc
                return carry2

            lax.fori_loop(0, groups, grp_body, 0)
            pltpu.sync_copy(oc_v, out_hbm.at[pl.ds(base, c)])
            return carry

        lax.fori_loop(0, n_chunks, chunk_body, 0)

    return edge_dot


def kernel(x, edge_index):
    n_nodes, d = x.shape
    e = edge_index.shape[1]
    norm_h = _normalize_tc(x)
    src = edge_index[0]
    dst = edge_index[1]
    nw = 32
    cos = _make_edge_dot(n_nodes, d, e, nw)(norm_h, src, dst)
    return cos.reshape(e, 1)


# speedup vs baseline: 3.0869x; 3.0869x over previous
import functools

import jax
import jax.numpy as jnp
from jax import lax
from jax.experimental import pallas as pl
from jax.experimental.pallas import tpu as pltpu
from jax.experimental.pallas import tpu_sc as plsc


def _normalize_body(x_ref, o_ref):
    v = x_ref[...]
    n = jnp.sqrt(jnp.sum(v * v, axis=1, keepdims=True))
    o_ref[...] = v / jnp.maximum(n, 1e-12)


def _normalize_tc(x):
    n, d = x.shape
    return pl.pallas_call(
        _normalize_body,
        out_shape=jax.ShapeDtypeStruct((n, d), jnp.float32),
    )(x)


_L = 16


def _make_edge_dot(n_nodes, d, e, nw):
    epw = e // nw
    c = 80
    assert epw % c == 0 and c % _L == 0 and c % 8 == 0
    n_chunks = epw // c
    groups = c // _L

    mesh = plsc.VectorSubcoreMesh(core_axis_name="c", subcore_axis_name="s")
    nc = 2

    @functools.partial(
        pl.kernel,
        mesh=mesh,
        out_type=jax.ShapeDtypeStruct((e,), jnp.float32),
        compiler_params=pltpu.CompilerParams(needs_layout_passes=False),
        scratch_types=[
            pltpu.VMEM((c,), jnp.int32),
            pltpu.VMEM((c,), jnp.int32),
            pltpu.VMEM((c, d), jnp.float32),
            pltpu.VMEM((c, d), jnp.float32),
            pltpu.VMEM((c,), jnp.float32),
            pltpu.SemaphoreType.DMA,
            pltpu.SemaphoreType.DMA,
        ],
    )
    def edge_dot(h_hbm, src_hbm, dst_hbm, out_hbm,
                 si_v, di_v, ru_v, rv_v, oc_v, sem_u, sem_v):
        wid = lax.axis_index("s") * nc + lax.axis_index("c")
        wbase = wid * epw

        def chunk_body(ci, carry):
            base = wbase + ci * c
            pltpu.sync_copy(src_hbm.at[pl.ds(base, c)], si_v)
            pltpu.sync_copy(dst_hbm.at[pl.ds(base, c)], di_v)
            cu = pltpu.async_copy(h_hbm.at[si_v], ru_v, sem_u)
            cv = pltpu.async_copy(h_hbm.at[di_v], rv_v, sem_v)
            cu.wait()
            cv.wait()

            lane = lax.broadcasted_iota(jnp.int32, (_L,), 0)
            last = lane == (_L - 1)

            def edge_body(e, carry2):
                parts = []
                for k in range(d // _L):
                    gu = ru_v[e, pl.ds(k * _L, _L)]
                    gv = rv_v[e, pl.ds(k * _L, _L)]
                    parts.append(gu * gv)
                while len(parts) > 1:
                    parts = [a + b for a, b in zip(parts[::2], parts[1::2])]
                cum = plsc.cumsum(parts[0])
                idx = jnp.full((_L,), e, dtype=jnp.int32)
                plsc.store_scatter(oc_v, [idx], cum, mask=last)
                return carry2

            lax.fori_loop(0, c, edge_body, 0)
            pltpu.sync_copy(oc_v, out_hbm.at[pl.ds(base, c)])
            return carry

        lax.fori_loop(0, n_chunks, chunk_body, 0)

    return edge_dot


def kernel(x, edge_index):
    n_nodes, d = x.shape
    e = edge_index.shape[1]
    norm_h = _normalize_tc(x)
    src = edge_index[0]
    dst = edge_index[1]
    nw = 32
    cos = _make_edge_dot(n_nodes, d, e, nw)(norm_h, src, dst)
    return cos.reshape(e, 1)


# speedup vs baseline: 5.9170x; 1.9168x over previous
import functools

import jax
import jax.numpy as jnp
from jax import lax
from jax.experimental import pallas as pl
from jax.experimental.pallas import tpu as pltpu
from jax.experimental.pallas import tpu_sc as plsc


def _normalize_body(x_ref, o_ref):
    v = x_ref[...]
    n = jnp.sqrt(jnp.sum(v * v, axis=1, keepdims=True))
    o_ref[...] = v / jnp.maximum(n, 1e-12)


def _normalize_tc(x):
    n, d = x.shape
    return pl.pallas_call(
        _normalize_body,
        out_shape=jax.ShapeDtypeStruct((n, d), jnp.float32),
    )(x)


_L = 16
_NW = 32
_C = 40
_NBUF = 5


def _make_edge_dot(n_nodes, d, e):
    epw = e // _NW
    n_chunks = epw // _C
    assert epw % _C == 0 and n_chunks % _NBUF == 0 and _C % 8 == 0

    mesh = plsc.VectorSubcoreMesh(core_axis_name="c", subcore_axis_name="s")
    nc = 2

    @functools.partial(
        pl.kernel,
        mesh=mesh,
        out_type=jax.ShapeDtypeStruct((e,), jnp.float32),
        compiler_params=pltpu.CompilerParams(needs_layout_passes=False),
        scratch_types=[
            pltpu.VMEM((n_chunks, _C), jnp.int32),
            pltpu.VMEM((n_chunks, _C), jnp.int32),
            pltpu.VMEM((_NBUF, _C, d), jnp.float32),
            pltpu.VMEM((_NBUF, _C, d), jnp.float32),
            pltpu.VMEM((epw,), jnp.float32),
        ] + [pltpu.SemaphoreType.DMA] * _NBUF,
    )
    def edge_dot(h_hbm, src_hbm, dst_hbm, out_hbm,
                 si_v, di_v, ru_v, rv_v, oc_v, *sems):
        wid = lax.axis_index("s") * nc + lax.axis_index("c")
        pltpu.sync_copy(src_hbm.at[wid], si_v)
        pltpu.sync_copy(dst_hbm.at[wid], di_v)

        def fire(b, chunk):
            pltpu.async_copy(h_hbm.at[si_v.at[chunk]], ru_v.at[b], sems[b])
            pltpu.async_copy(h_hbm.at[di_v.at[chunk]], rv_v.at[b], sems[b])

        def drain(b, chunk):
            pltpu.make_async_copy(
                h_hbm.at[si_v.at[chunk]], ru_v.at[b], sems[b]).wait()
            pltpu.make_async_copy(
                h_hbm.at[di_v.at[chunk]], rv_v.at[b], sems[b]).wait()

        lane = lax.broadcasted_iota(jnp.int32, (_L,), 0)
        last = lane == (_L - 1)

        def compute(b, chunk):
            ru = ru_v.at[b]
            rv = rv_v.at[b]

            def edge_body(eidx, carry):
                for u in range(2):
                    ei = eidx * 2 + u
                    parts = []
                    for k in range(d // _L):
                        gu = ru[ei, pl.ds(k * _L, _L)]
                        gv = rv[ei, pl.ds(k * _L, _L)]
                        parts.append(gu * gv)
                    while len(parts) > 1:
                        parts = [a2 + b2 for a2, b2 in
                                 zip(parts[::2], parts[1::2])]
                    cum = plsc.cumsum(parts[0])
                    ie = jnp.full((_L,), chunk * _C + ei, dtype=jnp.int32)
                    plsc.store_scatter(oc_v, [ie], cum, mask=last)
                return carry

            lax.fori_loop(0, _C // 2, edge_body, 0)

        for b in range(_NBUF):
            fire(b, b)

        def blk_body(blk, carry):
            for b in range(_NBUF):
                chunk = blk * _NBUF + b
                drain(b, chunk)
                compute(b, chunk)
                nxt = chunk + _NBUF

                @pl.when(nxt < n_chunks)
                def _():
                    fire(b, nxt)
            return carry

        lax.fori_loop(0, n_chunks // _NBUF, blk_body, 0)
        pltpu.sync_copy(oc_v, out_hbm.at[pl.ds(wid * epw, epw)])

    return edge_dot


def kernel(x, edge_index):
    n_nodes, d = x.shape
    e = edge_index.shape[1]
    norm_h = _normalize_tc(x)
    n_chunks = e // (_NW * _C)
    src3 = edge_index[0].reshape(_NW, n_chunks, _C)
    dst3 = edge_index[1].reshape(_NW, n_chunks, _C)
    cos = _make_edge_dot(n_nodes, d, e)(norm_h, src3, dst3)
    return cos.reshape(e, 1)


# speedup vs baseline: 6.2857x; 1.0623x over previous
import functools

import jax
import jax.numpy as jnp
from jax import lax
from jax.experimental import pallas as pl
from jax.experimental.pallas import tpu as pltpu
from jax.experimental.pallas import tpu_sc as plsc


def _normalize_body(x_ref, o_ref):
    v = x_ref[...]
    n = jnp.sqrt(jnp.sum(v * v, axis=1, keepdims=True))
    o_ref[...] = (v / jnp.maximum(n, 1e-12)).astype(jnp.bfloat16)


def _normalize_tc(x):
    n, d = x.shape
    return pl.pallas_call(
        _normalize_body,
        out_shape=jax.ShapeDtypeStruct((n, d), jnp.bfloat16),
    )(x)


_L = 16
_NW = 32
_C = 80
_NBUF = 5


def _make_edge_dot(n_nodes, d, e):
    epw = e // _NW
    n_chunks = epw // _C
    d2 = d // 2
    assert epw % _C == 0 and n_chunks % _NBUF == 0 and _C % 8 == 0

    mesh = plsc.VectorSubcoreMesh(core_axis_name="c", subcore_axis_name="s")
    nc = 2

    @functools.partial(
        pl.kernel,
        mesh=mesh,
        out_type=jax.ShapeDtypeStruct((e,), jnp.float32),
        compiler_params=pltpu.CompilerParams(
            needs_layout_passes=False, use_tc_tiling_on_sc=False),
        scratch_types=[
            pltpu.VMEM((n_chunks, _C), jnp.int32),
            pltpu.VMEM((n_chunks, _C), jnp.int32),
            pltpu.VMEM((_NBUF, _C, d2), jnp.float32),
            pltpu.VMEM((_NBUF, _C, d2), jnp.float32),
            pltpu.VMEM((epw,), jnp.float32),
        ] + [pltpu.SemaphoreType.DMA] * _NBUF,
    )
    def edge_dot(h_hbm, src_hbm, dst_hbm, out_hbm,
                 si_v, di_v, ru_v, rv_v, oc_v, *sems):
        wid = lax.axis_index("s") * nc + lax.axis_index("c")
        pltpu.sync_copy(src_hbm.at[wid], si_v)
        pltpu.sync_copy(dst_hbm.at[wid], di_v)

        def fire(b, chunk):
            pltpu.async_copy(h_hbm.at[si_v.at[chunk]], ru_v.at[b], sems[b])
            pltpu.async_copy(h_hbm.at[di_v.at[chunk]], rv_v.at[b], sems[b])

        def drain(b, chunk):
            pltpu.make_async_copy(
                h_hbm.at[si_v.at[chunk]], ru_v.at[b], sems[b]).wait()
            pltpu.make_async_copy(
                h_hbm.at[di_v.at[chunk]], rv_v.at[b], sems[b]).wait()

        lane = lax.broadcasted_iota(jnp.int32, (_L,), 0)
        last = lane == (_L - 1)

        def compute(b, chunk):
            ru = ru_v.at[b]
            rv = rv_v.at[b]

            def edge_body(eidx, carry):
                for u in range(2):
                    ei = eidx * 2 + u
                    parts = []
                    for k in range(d2 // _L):
                        gu = ru[ei, pl.ds(k * _L, _L)]
                        gv = rv[ei, pl.ds(k * _L, _L)]
                        bu = plsc.bitcast(gu, jnp.bfloat16)
                        bv = plsc.bitcast(gv, jnp.bfloat16)
                        p0, p1 = plsc.unpack(
                            bu * bv, format=plsc.PackFormat.INTERLEAVED)
                        parts.append(p0)
                        parts.append(p1)
                    while len(parts) > 1:
                        parts = [a2 + b2 for a2, b2 in
                                 zip(parts[::2], parts[1::2])]
                    cum = plsc.cumsum(parts[0])
                    ie = jnp.full((_L,), chunk * _C + ei, dtype=jnp.int32)
                    plsc.store_scatter(oc_v, [ie], cum, mask=last)
                return carry

            lax.fori_loop(0, _C // 2, edge_body, 0)

        for b in range(_NBUF):
            fire(b, b)

        def blk_body(blk, carry):
            for b in range(_NBUF):
                chunk = blk * _NBUF + b
                drain(b, chunk)
                compute(b, chunk)
                nxt = chunk + _NBUF

                @pl.when(nxt < n_chunks)
                def _():
                    fire(b, nxt)
            return carry

        lax.fori_loop(0, n_chunks // _NBUF, blk_body, 0)
        pltpu.sync_copy(oc_v, out_hbm.at[pl.ds(wid * epw, epw)])

    return edge_dot


def kernel(x, edge_index):
    n_nodes, d = x.shape
    e = edge_index.shape[1]
    norm_b = _normalize_tc(x)
    norm_h = jax.lax.bitcast_convert_type(
        norm_b.reshape(n_nodes, d // 2, 2), jnp.float32)
    n_chunks = e // (_NW * _C)
    src3 = edge_index[0].reshape(_NW, n_chunks, _C)
    dst3 = edge_index[1].reshape(_NW, n_chunks, _C)
    cos = _make_edge_dot(n_nodes, d, e)(norm_h, src3, dst3)
    return cos.reshape(e, 1)
